# baseline (speedup 1.0000x reference)
import functools

import jax
import jax.numpy as jnp
from jax import lax
from jax.experimental import pallas as pl
from jax.experimental.pallas import tpu as pltpu
from jax.experimental.pallas import tpu_sc as plsc

NUM_ITEMS = 100000
EMBED_DIM = 128
BATCH = 4096
HIST_LEN = 50

NC, NS, LANES = 2, 16, 16
NW = NC * NS
SPW = BATCH // NW
SGRP = 4
GIDX = SGRP * HIST_LEN
NG = SPW // SGRP
NCH = EMBED_DIM // LANES

TROWS = NUM_ITEMS + 1
TCH = 104
TK = 31
TBASE = TCH * TK
TREM = NUM_ITEMS - (NW - 1) * TBASE
TREMBASE = (NW - 1) * TBASE
NBUF = 4

_MESH = plsc.VectorSubcoreMesh(
    core_axis_name="c", subcore_axis_name="s", num_cores=NC, num_subcores=NS
)


@functools.partial(
    pl.kernel,
    out_type=(
        jax.ShapeDtypeStruct((BATCH, EMBED_DIM), jnp.float32),
        jax.ShapeDtypeStruct((TROWS, EMBED_DIM), jnp.float32),
    ),
    mesh=_MESH,
    scratch_types=[
        pltpu.VMEM((SPW * HIST_LEN,), jnp.int32),
        pltpu.VMEM((GIDX, EMBED_DIM), jnp.float32),
        pltpu.VMEM((GIDX, EMBED_DIM), jnp.float32),
        pltpu.VMEM((SPW, EMBED_DIM), jnp.float32),
        pltpu.VMEM_SHARED((NS * NBUF * TCH, EMBED_DIM), jnp.float32),
        pltpu.SemaphoreType.DMA,
        pltpu.SemaphoreType.DMA,
        pltpu.SemaphoreType.DMA,
        pltpu.SemaphoreType.DMA,
        pltpu.SemaphoreType.DMA,
        pltpu.SemaphoreType.DMA,
        pltpu.SemaphoreType.DMA,
        pltpu.SemaphoreType.DMA,
        pltpu.SemaphoreType.DMA,
        pltpu.SemaphoreType.DMA,
    ],
)
def _session_mean_sc(
    idx_hbm, table_hbm, out_hbm, tbl_out_hbm,
    idx_v, rows0, rows1, out_v, tshared,
    sem0, sem1, si0, si1, si2, si3, so0, so1, so2, so3,
):
    wid = lax.axis_index("s") * NC + lax.axis_index("c")
    sub = lax.axis_index("s")
    sis = (si0, si1, si2, si3)
    sos = (so0, so1, so2, so3)
    base = pl.multiple_of(wid * (SPW * HIST_LEN), 8)
    pltpu.sync_copy(idx_hbm.at[pl.ds(base, SPW * HIST_LEN)], idx_v)

    def _t_refs(k, b):
        boff = pl.multiple_of(sub * (NBUF * TCH) + b * TCH, 8)
        s = pl.multiple_of(wid * TBASE + k * TCH, 8)
        return (
            table_hbm.at[pl.ds(s, TCH)],
            tshared.at[pl.ds(boff, TCH)],
            tbl_out_hbm.at[pl.ds(s, TCH)],
        )

    def t_in(k, b):
        src, stage, _ = _t_refs(k, b)
        pltpu.async_copy(src, stage, sis[b])

    def t_in_wait(k, b):
        src, stage, _ = _t_refs(k, b)
        pltpu.make_async_copy(src, stage, sis[b]).wait()

    def t_out(k, b):
        _, stage, dst = _t_refs(k, b)
        pltpu.async_copy(stage, dst, sos[b])

    def t_out_wait(k, b):
        _, stage, dst = _t_refs(k, b)
        pltpu.make_async_copy(stage, dst, sos[b]).wait()

    GCUTS = (0, 56, 104, 160, GIDX)

    def issue(g, buf, sem):
        off = pl.multiple_of(g * GIDX, 8)
        for a, b in zip(GCUTS[:-1], GCUTS[1:]):
            pltpu.async_copy(
                table_hbm.at[idx_v.at[pl.ds(off + a, b - a)]],
                buf.at[pl.ds(a, b - a)],
                sem,
            )

    def wait(g, buf, sem):
        off = pl.multiple_of(g * GIDX, 8)
        for a, b in zip(GCUTS[:-1], GCUTS[1:]):
            pltpu.make_async_copy(
                table_hbm.at[idx_v.at[pl.ds(off + a, b - a)]],
                buf.at[pl.ds(a, b - a)],
                sem,
            ).wait()

    inv_l = jnp.float32(1.0 / HIST_LEN)

    def accumulate(g, buf):
        def jbody(j, accs):
            return tuple(
                accs[s * NCH + c] + buf[s * HIST_LEN + j, pl.ds(c * LANES, LANES)]
                for s in range(SGRP)
                for c in range(NCH)
            )

        accs = lax.fori_loop(
            0, HIST_LEN, jbody,
            tuple(jnp.zeros((LANES,), jnp.float32) for _ in range(SGRP * NCH)),
        )
        for s in range(SGRP):
            orow = g * SGRP + s
            for c in range(NCH):
                out_v[orow, pl.ds(c * LANES, LANES)] = accs[s * NCH + c] * inv_l

    issue(0, rows0, sem0)

    @pl.when(wid == NW - 1)
    def _():
        rboff = pl.multiple_of(sub * (NBUF * TCH), 8)
        pltpu.sync_copy(
            table_hbm.at[pl.ds(TREMBASE, TREM)],
            tshared.at[pl.ds(rboff, TREM)],
        )
        pltpu.sync_copy(
            tshared.at[pl.ds(rboff, TREM)],
            tbl_out_hbm.at[pl.ds(TREMBASE, TREM)],
        )

    @pl.when(wid < NW - 1)
    def _():
        for b in range(NBUF):
            t_in(b, b)

    def body(j, carry):
        g0 = j * 4
        k0 = j * 4

        for t in range(NBUF):
            @pl.when(jnp.logical_and(wid < NW - 1, k0 + t < TK))
            def _(t=t):
                t_in_wait(k0 + t, t)
                t_out(k0 + t, t)

        wait(g0, rows0, sem0)
        issue(g0 + 1, rows1, sem1)
        accumulate(g0, rows0)
        wait(g0 + 1, rows1, sem1)
        issue(g0 + 2, rows0, sem0)
        accumulate(g0 + 1, rows1)

        for t in range(NBUF):
            @pl.when(jnp.logical_and(wid < NW - 1, k0 + t < TK))
            def _(t=t):
                t_out_wait(k0 + t, t)

            @pl.when(jnp.logical_and(wid < NW - 1, k0 + t + NBUF < TK))
            def _(t=t):
                t_in(k0 + t + NBUF, t)

        wait(g0 + 2, rows0, sem0)
        issue(g0 + 3, rows1, sem1)
        accumulate(g0 + 2, rows0)
        wait(g0 + 3, rows1, sem1)

        @pl.when(j < NG // 4 - 1)
        def _():
            issue(g0 + 4, rows0, sem0)

        accumulate(g0 + 3, rows1)
        return carry

    lax.fori_loop(0, NG // 4, body, 0)

    obase = pl.multiple_of(wid * SPW, 8)
    pltpu.sync_copy(out_v, out_hbm.at[pl.ds(obase, SPW)])


def kernel(sess2items, pos_items, neg_items, item_embeddings):
    idx_flat = sess2items.astype(jnp.int32).reshape(-1)
    session_embedding, table_out = _session_mean_sc(idx_flat, item_embeddings)
    table_out = table_out.at[NUM_ITEMS].set(item_embeddings[NUM_ITEMS])
    return (session_embedding, table_out)


# speedup vs baseline: 1.0063x; 1.0063x over previous
import functools

import jax
import jax.numpy as jnp
from jax import lax
from jax.experimental import pallas as pl
from jax.experimental.pallas import tpu as pltpu
from jax.experimental.pallas import tpu_sc as plsc

NUM_ITEMS = 100000
EMBED_DIM = 128
BATCH = 4096
HIST_LEN = 50

NC, NS, LANES = 2, 16, 16
NW = NC * NS
SPW = BATCH // NW
SGRP = 4
GIDX = SGRP * HIST_LEN
NG = SPW // SGRP
SPLIT = 104
NCH = EMBED_DIM // LANES

TROWS = NUM_ITEMS + 1
TBASE = 3200
TCH = 128
T31BASE = (NW - 1) * TBASE
TCH31 = 32
TK = 25

_MESH = plsc.VectorSubcoreMesh(
    core_axis_name="c", subcore_axis_name="s", num_cores=NC, num_subcores=NS
)


@functools.partial(
    pl.kernel,
    out_type=(
        jax.ShapeDtypeStruct((BATCH, EMBED_DIM), jnp.float32),
        jax.ShapeDtypeStruct((TROWS, EMBED_DIM), jnp.float32),
    ),
    mesh=_MESH,
    scratch_types=[
        pltpu.VMEM((SPW * HIST_LEN,), jnp.int32),
        pltpu.VMEM((GIDX, EMBED_DIM), jnp.float32),
        pltpu.VMEM((GIDX, EMBED_DIM), jnp.float32),
        pltpu.VMEM((SPW, EMBED_DIM), jnp.float32),
        pltpu.VMEM_SHARED((NS * 2 * TCH, EMBED_DIM), jnp.float32),
        pltpu.SemaphoreType.DMA,
        pltpu.SemaphoreType.DMA,
        pltpu.SemaphoreType.DMA,
        pltpu.SemaphoreType.DMA,
        pltpu.SemaphoreType.DMA,
        pltpu.SemaphoreType.DMA,
    ],
)
def _session_mean_sc(
    idx_hbm, table_hbm, out_hbm, tbl_out_hbm,
    idx_v, rows0, rows1, out_v, tshared,
    sem0, sem1, si0, si1, so0, so1,
):
    wid = lax.axis_index("s") * NC + lax.axis_index("c")
    sub = lax.axis_index("s")
    base = pl.multiple_of(wid * (SPW * HIST_LEN), 8)
    pltpu.sync_copy(idx_hbm.at[pl.ds(base, SPW * HIST_LEN)], idx_v)

    def t_in(k, b, sem):
        boff = pl.multiple_of(sub * (2 * TCH) + b * TCH, 8)

        @pl.when(wid < NW - 1)
        def _():
            s = pl.multiple_of(wid * TBASE + k * TCH, 8)
            pltpu.async_copy(
                table_hbm.at[pl.ds(s, TCH)], tshared.at[pl.ds(boff, TCH)], sem
            )

        @pl.when(wid == NW - 1)
        def _():
            s = pl.multiple_of(T31BASE + k * TCH31, 8)
            pltpu.async_copy(
                table_hbm.at[pl.ds(s, TCH31)], tshared.at[pl.ds(boff, TCH31)], sem
            )

    def t_in_wait(k, b, sem):
        boff = pl.multiple_of(sub * (2 * TCH) + b * TCH, 8)

        @pl.when(wid < NW - 1)
        def _():
            s = pl.multiple_of(wid * TBASE + k * TCH, 8)
            pltpu.make_async_copy(
                table_hbm.at[pl.ds(s, TCH)], tshared.at[pl.ds(boff, TCH)], sem
            ).wait()

        @pl.when(wid == NW - 1)
        def _():
            s = pl.multiple_of(T31BASE + k * TCH31, 8)
            pltpu.make_async_copy(
                table_hbm.at[pl.ds(s, TCH31)], tshared.at[pl.ds(boff, TCH31)], sem
            ).wait()

    def t_out(k, b, sem):
        boff = pl.multiple_of(sub * (2 * TCH) + b * TCH, 8)

        @pl.when(wid < NW - 1)
        def _():
            s = pl.multiple_of(wid * TBASE + k * TCH, 8)
            pltpu.async_copy(
                tshared.at[pl.ds(boff, TCH)], tbl_out_hbm.at[pl.ds(s, TCH)], sem
            )

        @pl.when(wid == NW - 1)
        def _():
            s = pl.multiple_of(T31BASE + k * TCH31, 8)
            pltpu.async_copy(
                tshared.at[pl.ds(boff, TCH31)], tbl_out_hbm.at[pl.ds(s, TCH31)], sem
            )

    def t_out_wait(k, b, sem):
        boff = pl.multiple_of(sub * (2 * TCH) + b * TCH, 8)

        @pl.when(wid < NW - 1)
        def _():
            s = pl.multiple_of(wid * TBASE + k * TCH, 8)
            pltpu.make_async_copy(
                tshared.at[pl.ds(boff, TCH)], tbl_out_hbm.at[pl.ds(s, TCH)], sem
            ).wait()

        @pl.when(wid == NW - 1)
        def _():
            s = pl.multiple_of(T31BASE + k * TCH31, 8)
            pltpu.make_async_copy(
                tshared.at[pl.ds(boff, TCH31)], tbl_out_hbm.at[pl.ds(s, TCH31)], sem
            ).wait()

    GCUTS = (0, 56, 104, 160, GIDX)

    def issue(g, buf, sem):
        off = pl.multiple_of(g * GIDX, 8)
        for a, b in zip(GCUTS[:-1], GCUTS[1:]):
            pltpu.async_copy(
                table_hbm.at[idx_v.at[pl.ds(off + a, b - a)]],
                buf.at[pl.ds(a, b - a)],
                sem,
            )

    def wait(g, buf, sem):
        off = pl.multiple_of(g * GIDX, 8)
        for a, b in zip(GCUTS[:-1], GCUTS[1:]):
            pltpu.make_async_copy(
                table_hbm.at[idx_v.at[pl.ds(off + a, b - a)]],
                buf.at[pl.ds(a, b - a)],
                sem,
            ).wait()

    inv_l = jnp.float32(1.0 / HIST_LEN)

    def accumulate(g, buf):
        def jbody(j, accs):
            return tuple(
                accs[s * NCH + c] + buf[s * HIST_LEN + j, pl.ds(c * LANES, LANES)]
                for s in range(SGRP)
                for c in range(NCH)
            )

        accs = lax.fori_loop(
            0, HIST_LEN, jbody,
            tuple(jnp.zeros((LANES,), jnp.float32) for _ in range(SGRP * NCH)),
        )
        for s in range(SGRP):
            orow = g * SGRP + s
            for c in range(NCH):
                out_v[orow, pl.ds(c * LANES, LANES)] = accs[s * NCH + c] * inv_l

    issue(0, rows0, sem0)

    def body(i, carry):
        g0 = i * 2
        k0 = i * 2
        k1 = i * 2 + 1

        @pl.when(jnp.logical_and(k0 >= 2, k0 - 2 < TK))
        def _():
            t_out_wait(k0 - 2, 0, so0)

        @pl.when(jnp.logical_and(k1 >= 2, k1 - 2 < TK))
        def _():
            t_out_wait(k1 - 2, 1, so1)

        @pl.when(k0 < TK)
        def _():
            t_in(k0, 0, si0)

        @pl.when(k1 < TK)
        def _():
            t_in(k1, 1, si1)

        wait(g0, rows0, sem0)
        issue(g0 + 1, rows1, sem1)
        accumulate(g0, rows0)
        wait(g0 + 1, rows1, sem1)

        @pl.when(i < NG // 2 - 1)
        def _():
            issue(g0 + 2, rows0, sem0)

        accumulate(g0 + 1, rows1)

        @pl.when(k0 < TK)
        def _():
            t_in_wait(k0, 0, si0)
            t_out(k0, 0, so0)

        @pl.when(k1 < TK)
        def _():
            t_in_wait(k1, 1, si1)
            t_out(k1, 1, so1)

        return carry

    lax.fori_loop(0, NG // 2, body, 0)

    obase = pl.multiple_of(wid * SPW, 8)
    pltpu.sync_copy(out_v, out_hbm.at[pl.ds(obase, SPW)])


def kernel(sess2items, pos_items, neg_items, item_embeddings):
    idx_flat = sess2items.astype(jnp.int32).reshape(-1)
    session_embedding, table_out = _session_mean_sc(idx_flat, item_embeddings)
    table_out = table_out.at[NUM_ITEMS].set(item_embeddings[NUM_ITEMS])
    return (session_embedding, table_out)
